# baseline (speedup 1.0000x reference)
import jax
import jax.numpy as jnp
from jax import lax
from jax.experimental import pallas as pl
from jax.experimental.pallas import tpu as pltpu
from jax.experimental.pallas import tpu_sc as plsc

Z = 0.8
N = 10000
D = 256
H = 128
E = 160000
NSUB = 5000

NS = 16
CB = 128
CH = 80
WIN = 8
EP = NS * CH * CB
NPAD = 10240
RPT = NPAD // NS

BLK = 1000


def _mask_body(row_ref, col_ref, m_ref):
    r = row_ref[...]
    c = col_ref[...]
    ri = lax.broadcasted_iota(jnp.int32, (NSUB, D), 1)
    R = (r == ri).astype(jnp.float32)
    C = (c == ri).astype(jnp.float32)
    cnt = lax.dot_general(R, C, (((0,), (0,)), ((), ())),
                          preferred_element_type=jnp.float32)
    m_ref[...] = (cnt > 0.5).astype(jnp.float32)


def _build_mask(rs, cs):
    return pl.pallas_call(
        _mask_body,
        out_shape=jax.ShapeDtypeStruct((D, D), jnp.float32),
    )(rs, cs)


def _t_body(x_ref, w0_ref, w1_ref, b0_ref, b1_ref, m_ref, out_ref):
    i = pl.program_id(1)
    x = x_ref[...]
    x0 = jnp.maximum(x @ w0_ref[...] + b0_ref[...], 0.0)
    x1 = jnp.maximum(x @ w1_ref[...] + b1_ref[...], 0.0)
    a = Z * x0 + (1.0 - Z) * x1
    dlt = (2.0 * Z - 1.0) * (x1 - x0)
    m = jnp.concatenate(
        [m_ref[...], jnp.zeros((BLK - D, H), jnp.float32)], axis=0)
    m = jnp.where(i == 0, m, 0.0)
    out_ref[...] = a + m * dlt


def _transform(x, w0, w1, b0, b1, mask):
    grid = (2, N // BLK)
    return pl.pallas_call(
        _t_body,
        grid=grid,
        in_specs=[
            pl.BlockSpec((BLK, D), lambda h, i: (i, 0)),
            pl.BlockSpec((D, H), lambda h, i: (0, h)),
            pl.BlockSpec((D, H), lambda h, i: (0, h)),
            pl.BlockSpec((1, H), lambda h, i: (0, h)),
            pl.BlockSpec((1, H), lambda h, i: (0, h)),
            pl.BlockSpec((D, H), lambda h, i: (0, h)),
        ],
        out_specs=pl.BlockSpec(
            (BLK, H), lambda h, i: (h * (N // BLK) + i, 0)),
        out_shape=jax.ShapeDtypeStruct((2 * N, H), jnp.float32),
    )(x, w0, w1, b0, b1, mask)


def _scale_chunk(buf, wv, srcv, histv, j, o16, g_lo=0, g_hi=CB // 16):
    def group(g, carry):
        wv16 = wv[j, pl.ds(g * 16, 16)]
        idx16 = srcv[j, pl.ds(g * 16, 16)]
        plsc.addupdate_scatter(
            histv, [lax.shift_right_logical(idx16, 7),
                    lax.bitwise_and(idx16, 127)], o16)
        for lane in range(16):
            wb = jnp.full((16,), wv16[lane], jnp.float32)
            e = g * 16 + lane
            for k in range(H // 16):
                sl = buf[e, pl.ds(k * 16, 16)]
                buf[e, pl.ds(k * 16, 16)] = sl * wb
        return carry

    lax.fori_loop(g_lo, g_hi, group, 0)


def _spmm_body(xstack, src3, dst3, w3,
               slo_out, shi_out,
               srcv, dstv, wv, rows, rows1, histv, idx80,
               sem, sem1, sems, sems1,
               shared_sum, shared_hist):
    c = lax.axis_index("c")
    s = lax.axis_index("s")

    z16 = jnp.zeros((16,), jnp.float32)
    o16 = jnp.ones((16,), jnp.float32)
    SR = NPAD // H // NS

    def zfill(r, carry):
        for k in range(H // 16):
            rows[r, pl.ds(k * 16, 16)] = z16
        return carry

    lax.fori_loop(0, CB, zfill, 0)

    def zhist(r, carry):
        for k in range(H // 16):
            histv[r, pl.ds(k * 16, 16)] = z16
        return carry

    lax.fori_loop(0, NPAD // H, zhist, 0)

    iota16 = lax.iota(jnp.int32, 16)
    for g in range(NPAD // H // 16):
        idx80[pl.ds(g * 16, 16)] = iota16 + (g * 16)

    for t in range(RPT // CB):
        pltpu.sync_copy(rows, shared_sum.at[pl.ds(s * RPT + t * CB, CB)])

    @pl.when(s < 10)
    def _():
        pltpu.sync_copy(rows.at[pl.ds(0, 8)], shared_hist.at[pl.ds(s * 8, 8)])

    plsc.subcore_barrier()

    offv = jnp.full((16,), c * N, jnp.int32)

    def window(jj, carry0):
        wsl = pl.ds(jj * WIN, WIN)
        pltpu.sync_copy(src3.at[s, wsl], srcv)
        pltpu.sync_copy(dst3.at[s, wsl], dstv)
        pltpu.sync_copy(w3.at[s, wsl], wv)

        def dadj(r, carry):
            for k in range(CB // 16):
                dstv[r, pl.ds(k * 16, 16)] = (
                    dstv[r, pl.ds(k * 16, 16)] + offv)
            return carry

        lax.fori_loop(0, WIN, dadj, 0)

        def pair(p, carry):
            j0 = 2 * p
            j1 = 2 * p + 1

            @pl.when(p == 0)
            def _():
                pltpu.async_copy(xstack.at[dstv.at[j0]], rows, sem)

            pltpu.make_async_copy(xstack.at[dstv.at[j0]], rows, sem).wait()
            _scale_chunk(rows, wv, srcv, histv, j0, o16, 0, CB // 32)

            @pl.when(jj + p > 0)
            def _():
                pltpu.make_async_copy(
                    rows1, shared_sum.at[srcv.at[j1]], sems1).wait()

            d1 = pltpu.async_copy(xstack.at[dstv.at[j1]], rows1, sem1)
            _scale_chunk(rows, wv, srcv, histv, j0, o16, CB // 32, CB // 16)
            sc0 = pltpu.async_copy(
                rows, shared_sum.at[srcv.at[j0]], sems, add=True)
            d1.wait()
            _scale_chunk(rows1, wv, srcv, histv, j1, o16)
            sc0.wait()

            @pl.when(p < WIN // 2 - 1)
            def _():
                pltpu.async_copy(xstack.at[dstv.at[j0 + 2]], rows, sem)

            pltpu.async_copy(rows1, shared_sum.at[srcv.at[j1]], sems1,
                             add=True)
            return carry

        lax.fori_loop(0, WIN // 2, pair, 0)
        return carry0

    lax.fori_loop(0, CH // WIN, window, 0)

    pltpu.make_async_copy(
        rows1, shared_sum.at[srcv.at[WIN - 1]], sems1).wait()

    pltpu.sync_copy(histv, shared_hist.at[idx80], add=True)
    plsc.subcore_barrier()
    pltpu.sync_copy(shared_hist.at[pl.ds(s * SR, SR)],
                    histv.at[pl.ds(0, SR)])

    def copyout(t, carry0):
        sl = pl.ds(s * RPT + t * CB, CB)
        pltpu.sync_copy(shared_sum.at[sl], rows)

        def divgroup(g, carry):
            cnt16 = histv[t, pl.ds(g * 16, 16)]
            rec16 = 1.0 / jnp.maximum(cnt16, 1.0)
            for lane in range(16):
                rb = jnp.full((16,), rec16[lane], jnp.float32)
                r = g * 16 + lane
                for k in range(H // 16):
                    rows[r, pl.ds(k * 16, 16)] = (
                        rows[r, pl.ds(k * 16, 16)] * rb)
            return carry

        lax.fori_loop(0, CB // 16, divgroup, 0)

        @pl.when(c == 0)
        def _():
            pltpu.sync_copy(rows, slo_out.at[sl])

        @pl.when(c == 1)
        def _():
            pltpu.sync_copy(rows, shi_out.at[sl])

        return carry0

    lax.fori_loop(0, RPT // CB, copyout, 0)


def _spmm(xstack, src3, dst3, w3):
    mesh = plsc.VectorSubcoreMesh(core_axis_name="c", subcore_axis_name="s")
    f = pl.kernel(
        _spmm_body,
        out_type=[
            jax.ShapeDtypeStruct((NPAD, H), jnp.float32),
            jax.ShapeDtypeStruct((NPAD, H), jnp.float32),
        ],
        mesh=mesh,
        compiler_params=pltpu.CompilerParams(needs_layout_passes=False),
        scratch_types=[
            pltpu.VMEM((WIN, CB), jnp.int32),
            pltpu.VMEM((WIN, CB), jnp.int32),
            pltpu.VMEM((WIN, CB), jnp.float32),
            pltpu.VMEM((CB, H), jnp.float32),
            pltpu.VMEM((CB, H), jnp.float32),
            pltpu.VMEM((NPAD // H, H), jnp.float32),
            pltpu.VMEM((NPAD // H,), jnp.int32),
            pltpu.SemaphoreType.DMA,
            pltpu.SemaphoreType.DMA,
            pltpu.SemaphoreType.DMA,
            pltpu.SemaphoreType.DMA,
            pltpu.VMEM_SHARED((NPAD, H), jnp.float32),
            pltpu.VMEM_SHARED((NPAD // H, H), jnp.float32),
        ],
    )
    return f(xstack, src3, dst3, w3)


def _c_body(slo_ref, shi_ref, x_ref, w0_ref, w1_ref,
            b0_ref, b1_ref, g_ref, be_ref, m_ref, o_ref):
    i = pl.program_id(0)
    xa = jnp.concatenate([slo_ref[...], shi_ref[...]], axis=1)
    mu = jnp.mean(xa, axis=1, keepdims=True)
    var = jnp.mean(jnp.square(xa - mu), axis=1, keepdims=True)
    xn = (xa - mu) * lax.rsqrt(var + 1e-5) * g_ref[...] + be_ref[...]
    x = x_ref[...]
    w0 = w0_ref[...]
    w1 = w1_ref[...]
    h0 = xn @ w0[:D] + x @ w0[D:] + b0_ref[...]
    h1 = xn @ w1[:D] + x @ w1[D:] + b1_ref[...]
    a = Z * h0 + (1.0 - Z) * h1
    dlt = (2.0 * Z - 1.0) * (h1 - h0)
    m = jnp.concatenate(
        [m_ref[...], jnp.zeros((BLK - D, D), jnp.float32)], axis=0)
    m = jnp.where(i == 0, m, 0.0)
    o_ref[...] = a + m * dlt


def _combine(slo, shi, x, w0, w1, b0, b1, g, be, mask):
    grid = (N // BLK,)
    return pl.pallas_call(
        _c_body,
        grid=grid,
        in_specs=[
            pl.BlockSpec((BLK, H), lambda i: (i, 0)),
            pl.BlockSpec((BLK, H), lambda i: (i, 0)),
            pl.BlockSpec((BLK, D), lambda i: (i, 0)),
            pl.BlockSpec((2 * D, D), lambda i: (0, 0)),
            pl.BlockSpec((2 * D, D), lambda i: (0, 0)),
            pl.BlockSpec((1, D), lambda i: (0, 0)),
            pl.BlockSpec((1, D), lambda i: (0, 0)),
            pl.BlockSpec((1, D), lambda i: (0, 0)),
            pl.BlockSpec((1, D), lambda i: (0, 0)),
            pl.BlockSpec((D, D), lambda i: (0, 0)),
        ],
        out_specs=pl.BlockSpec((BLK, D), lambda i: (i, 0)),
        out_shape=jax.ShapeDtypeStruct((N, D), jnp.float32),
    )(slo, shi, x, w0, w1, b0, b1, g, be, mask)


def kernel(x_, edge_index, edge_weight, node2subg,
           W_t0, b_t0, W_t1, b_t1, W_c0, b_c0, W_c1, b_c1, ln_g, ln_b):
    x_ = x_.astype(jnp.float32)
    src = edge_index[0].astype(jnp.int32)
    dst = edge_index[1].astype(jnp.int32)
    w = edge_weight.astype(jnp.float32)
    rs = node2subg[0].astype(jnp.int32).reshape(NSUB, 1)
    cs = node2subg[1].astype(jnp.int32).reshape(NSUB, 1)

    npad = EP - E
    ar = jnp.arange(npad, dtype=jnp.int32)
    src_p = jnp.concatenate([src, N + (ar % (NPAD - N))]).reshape(NS, CH, CB)
    dst_p = jnp.concatenate([dst, (ar * 997) % N]).reshape(NS, CH, CB)
    w_p = jnp.concatenate([w, jnp.zeros((npad,), jnp.float32)]
                          ).reshape(NS, CH, CB)

    mask = _build_mask(rs, cs)
    xstack = _transform(x_, W_t0, W_t1,
                        b_t0.reshape(1, D), b_t1.reshape(1, D), mask)
    slo, shi = _spmm(xstack, src_p, dst_p, w_p)
    out = _combine(slo, shi, x_, W_c0, W_c1,
                   b_c0.reshape(1, D), b_c1.reshape(1, D),
                   ln_g.reshape(1, D), ln_b.reshape(1, D), mask)
    return out


# speedup vs baseline: 1.0829x; 1.0829x over previous
import jax
import jax.numpy as jnp
from jax import lax
from jax.experimental import pallas as pl
from jax.experimental.pallas import tpu as pltpu
from jax.experimental.pallas import tpu_sc as plsc

Z = 0.8
N = 10000
D = 256
H = 128
E = 160000
NSUB = 5000

NS = 16
CB = 128
CH = 80
WIN = 8
EP = NS * CH * CB
NPAD = 10240
RPT = NPAD // NS

BLK = 1000


def _mask_body(row_ref, col_ref, m_ref):
    r = row_ref[...]
    c = col_ref[...]
    ri = lax.broadcasted_iota(jnp.int32, (NSUB, D), 1)
    R = (r == ri).astype(jnp.float32)
    C = (c == ri).astype(jnp.float32)
    cnt = lax.dot_general(R, C, (((0,), (0,)), ((), ())),
                          preferred_element_type=jnp.float32)
    m_ref[...] = (cnt > 0.5).astype(jnp.float32)


def _build_mask(rs, cs):
    return pl.pallas_call(
        _mask_body,
        out_shape=jax.ShapeDtypeStruct((D, D), jnp.float32),
    )(rs, cs)


def _t_body(x_ref, w0_ref, w1_ref, b0_ref, b1_ref, m_ref, out_ref):
    i = pl.program_id(1)
    x = x_ref[...]
    x0 = jnp.maximum(x @ w0_ref[...] + b0_ref[...], 0.0)
    x1 = jnp.maximum(x @ w1_ref[...] + b1_ref[...], 0.0)
    a = Z * x0 + (1.0 - Z) * x1
    dlt = (2.0 * Z - 1.0) * (x1 - x0)
    m = jnp.concatenate(
        [m_ref[...], jnp.zeros((BLK - D, H), jnp.float32)], axis=0)
    m = jnp.where(i == 0, m, 0.0)
    out_ref[...] = a + m * dlt


def _transform(x, w0, w1, b0, b1, mask):
    grid = (2, N // BLK)
    return pl.pallas_call(
        _t_body,
        grid=grid,
        in_specs=[
            pl.BlockSpec((BLK, D), lambda h, i: (i, 0)),
            pl.BlockSpec((D, H), lambda h, i: (0, h)),
            pl.BlockSpec((D, H), lambda h, i: (0, h)),
            pl.BlockSpec((1, H), lambda h, i: (0, h)),
            pl.BlockSpec((1, H), lambda h, i: (0, h)),
            pl.BlockSpec((D, H), lambda h, i: (0, h)),
        ],
        out_specs=pl.BlockSpec(
            (BLK, H), lambda h, i: (h * (N // BLK) + i, 0)),
        out_shape=jax.ShapeDtypeStruct((2 * N, H), jnp.float32),
    )(x, w0, w1, b0, b1, mask)


def _scale_chunk(buf, wv, srcv, histv, j, o16, g_lo=0, g_hi=CB // 16):
    def group(g, carry):
        wv16 = wv[j, pl.ds(g * 16, 16)]
        idx16 = srcv[j, pl.ds(g * 16, 16)]
        plsc.addupdate_scatter(
            histv, [lax.shift_right_logical(idx16, 7),
                    lax.bitwise_and(idx16, 127)], o16)
        for lane in range(16):
            wb = jnp.full((16,), wv16[lane], jnp.float32)
            e = g * 16 + lane
            for k in range(H // 16):
                sl = buf[e, pl.ds(k * 16, 16)]
                buf[e, pl.ds(k * 16, 16)] = sl * wb
        return carry

    lax.fori_loop(g_lo, g_hi, group, 0)


def _spmm_body(xstack, src3, dst3, w3,
               slo_out, shi_out,
               srcv, dstv, wv, rows, rows1, histv, idx80,
               sem, sem1, sems, sems1,
               shared_sum, shared_hist):
    c = lax.axis_index("c")
    s = lax.axis_index("s")

    z16 = jnp.zeros((16,), jnp.float32)
    o16 = jnp.ones((16,), jnp.float32)
    SR = NPAD // H // NS

    def zfill(r, carry):
        for k in range(H // 16):
            rows[r, pl.ds(k * 16, 16)] = z16
        return carry

    lax.fori_loop(0, CB, zfill, 0)

    def zhist(r, carry):
        for k in range(H // 16):
            histv[r, pl.ds(k * 16, 16)] = z16
        return carry

    lax.fori_loop(0, NPAD // H, zhist, 0)

    iota16 = lax.iota(jnp.int32, 16)
    for g in range(NPAD // H // 16):
        idx80[pl.ds(g * 16, 16)] = iota16 + (g * 16)

    for t in range(RPT // CB):
        pltpu.sync_copy(rows, shared_sum.at[pl.ds(s * RPT + t * CB, CB)])

    @pl.when(s < 10)
    def _():
        pltpu.sync_copy(rows.at[pl.ds(0, 8)], shared_hist.at[pl.ds(s * 8, 8)])

    plsc.subcore_barrier()

    offv = jnp.full((16,), c * N, jnp.int32)

    def window(jj, carry0):
        wsl = pl.ds(jj * WIN, WIN)
        pltpu.sync_copy(src3.at[s, wsl], srcv)
        pltpu.sync_copy(dst3.at[s, wsl], dstv)
        pltpu.sync_copy(w3.at[s, wsl], wv)

        def dadj(r, carry):
            for k in range(CB // 16):
                dstv[r, pl.ds(k * 16, 16)] = (
                    dstv[r, pl.ds(k * 16, 16)] + offv)
            return carry

        lax.fori_loop(0, WIN, dadj, 0)

        def pair(p, carry):
            j0 = 2 * p
            j1 = 2 * p + 1

            @pl.when(jj + p > 0)
            def _():
                pltpu.make_async_copy(
                    rows1, shared_sum.at[srcv.at[j1]], sems1).wait()

            @pl.when(p == 0)
            def _():
                pltpu.async_copy(xstack.at[dstv.at[j0]], rows, sem)

            d1 = pltpu.async_copy(xstack.at[dstv.at[j1]], rows1, sem1)
            pltpu.make_async_copy(xstack.at[dstv.at[j0]], rows, sem).wait()
            _scale_chunk(rows, wv, srcv, histv, j0, o16)
            sc0 = pltpu.async_copy(
                rows, shared_sum.at[srcv.at[j0]], sems, add=True)
            d1.wait()
            _scale_chunk(rows1, wv, srcv, histv, j1, o16)
            sc0.wait()

            @pl.when(p < WIN // 2 - 1)
            def _():
                pltpu.async_copy(xstack.at[dstv.at[j0 + 2]], rows, sem)

            pltpu.async_copy(rows1, shared_sum.at[srcv.at[j1]], sems1,
                             add=True)
            return carry

        lax.fori_loop(0, WIN // 2, pair, 0)
        return carry0

    lax.fori_loop(0, CH // WIN, window, 0)

    pltpu.make_async_copy(
        rows1, shared_sum.at[srcv.at[WIN - 1]], sems1).wait()

    pltpu.sync_copy(histv, shared_hist.at[idx80], add=True)
    plsc.subcore_barrier()
    pltpu.sync_copy(shared_hist.at[pl.ds(s * SR, SR)],
                    histv.at[pl.ds(0, SR)])

    def copyout(t, carry0):
        sl = pl.ds(s * RPT + t * CB, CB)
        pltpu.sync_copy(shared_sum.at[sl], rows)

        def divgroup(g, carry):
            cnt16 = histv[t, pl.ds(g * 16, 16)]
            rec16 = 1.0 / jnp.maximum(cnt16, 1.0)
            for lane in range(16):
                rb = jnp.full((16,), rec16[lane], jnp.float32)
                r = g * 16 + lane
                for k in range(H // 16):
                    rows[r, pl.ds(k * 16, 16)] = (
                        rows[r, pl.ds(k * 16, 16)] * rb)
            return carry

        lax.fori_loop(0, CB // 16, divgroup, 0)

        @pl.when(c == 0)
        def _():
            pltpu.sync_copy(rows, slo_out.at[sl])

        @pl.when(c == 1)
        def _():
            pltpu.sync_copy(rows, shi_out.at[sl])

        return carry0

    lax.fori_loop(0, RPT // CB, copyout, 0)


def _spmm(xstack, src3, dst3, w3):
    mesh = plsc.VectorSubcoreMesh(core_axis_name="c", subcore_axis_name="s")
    f = pl.kernel(
        _spmm_body,
        out_type=[
            jax.ShapeDtypeStruct((NPAD, H), jnp.float32),
            jax.ShapeDtypeStruct((NPAD, H), jnp.float32),
        ],
        mesh=mesh,
        compiler_params=pltpu.CompilerParams(needs_layout_passes=False),
        scratch_types=[
            pltpu.VMEM((WIN, CB), jnp.int32),
            pltpu.VMEM((WIN, CB), jnp.int32),
            pltpu.VMEM((WIN, CB), jnp.float32),
            pltpu.VMEM((CB, H), jnp.float32),
            pltpu.VMEM((CB, H), jnp.float32),
            pltpu.VMEM((NPAD // H, H), jnp.float32),
            pltpu.VMEM((NPAD // H,), jnp.int32),
            pltpu.SemaphoreType.DMA,
            pltpu.SemaphoreType.DMA,
            pltpu.SemaphoreType.DMA,
            pltpu.SemaphoreType.DMA,
            pltpu.VMEM_SHARED((NPAD, H), jnp.float32),
            pltpu.VMEM_SHARED((NPAD // H, H), jnp.float32),
        ],
    )
    return f(xstack, src3, dst3, w3)


def _c_body(slo_ref, shi_ref, x_ref, w0_ref, w1_ref,
            b0_ref, b1_ref, g_ref, be_ref, m_ref, o_ref):
    i = pl.program_id(0)
    xa = jnp.concatenate([slo_ref[...], shi_ref[...]], axis=1)
    mu = jnp.mean(xa, axis=1, keepdims=True)
    var = jnp.mean(jnp.square(xa - mu), axis=1, keepdims=True)
    xn = (xa - mu) * lax.rsqrt(var + 1e-5) * g_ref[...] + be_ref[...]
    x = x_ref[...]
    w0 = w0_ref[...]
    w1 = w1_ref[...]
    h0 = xn @ w0[:D] + x @ w0[D:] + b0_ref[...]
    h1 = xn @ w1[:D] + x @ w1[D:] + b1_ref[...]
    a = Z * h0 + (1.0 - Z) * h1
    dlt = (2.0 * Z - 1.0) * (h1 - h0)
    m = jnp.concatenate(
        [m_ref[...], jnp.zeros((BLK - D, D), jnp.float32)], axis=0)
    m = jnp.where(i == 0, m, 0.0)
    o_ref[...] = a + m * dlt


def _combine(slo, shi, x, w0, w1, b0, b1, g, be, mask):
    grid = (N // BLK,)
    return pl.pallas_call(
        _c_body,
        grid=grid,
        in_specs=[
            pl.BlockSpec((BLK, H), lambda i: (i, 0)),
            pl.BlockSpec((BLK, H), lambda i: (i, 0)),
            pl.BlockSpec((BLK, D), lambda i: (i, 0)),
            pl.BlockSpec((2 * D, D), lambda i: (0, 0)),
            pl.BlockSpec((2 * D, D), lambda i: (0, 0)),
            pl.BlockSpec((1, D), lambda i: (0, 0)),
            pl.BlockSpec((1, D), lambda i: (0, 0)),
            pl.BlockSpec((1, D), lambda i: (0, 0)),
            pl.BlockSpec((1, D), lambda i: (0, 0)),
            pl.BlockSpec((D, D), lambda i: (0, 0)),
        ],
        out_specs=pl.BlockSpec((BLK, D), lambda i: (i, 0)),
        out_shape=jax.ShapeDtypeStruct((N, D), jnp.float32),
    )(slo, shi, x, w0, w1, b0, b1, g, be, mask)


def kernel(x_, edge_index, edge_weight, node2subg,
           W_t0, b_t0, W_t1, b_t1, W_c0, b_c0, W_c1, b_c1, ln_g, ln_b):
    x_ = x_.astype(jnp.float32)
    src = edge_index[0].astype(jnp.int32)
    dst = edge_index[1].astype(jnp.int32)
    w = edge_weight.astype(jnp.float32)
    rs = node2subg[0].astype(jnp.int32).reshape(NSUB, 1)
    cs = node2subg[1].astype(jnp.int32).reshape(NSUB, 1)

    npad = EP - E
    ar = jnp.arange(npad, dtype=jnp.int32)
    src_p = jnp.concatenate([src, N + (ar % (NPAD - N))]).reshape(NS, CH, CB)
    dst_p = jnp.concatenate([dst, (ar * 997) % N]).reshape(NS, CH, CB)
    w_p = jnp.concatenate([w, jnp.zeros((npad,), jnp.float32)]
                          ).reshape(NS, CH, CB)

    mask = _build_mask(rs, cs)
    xstack = _transform(x_, W_t0, W_t1,
                        b_t0.reshape(1, D), b_t1.reshape(1, D), mask)
    slo, shi = _spmm(xstack, src_p, dst_p, w_p)
    out = _combine(slo, shi, x_, W_c0, W_c1,
                   b_c0.reshape(1, D), b_c1.reshape(1, D),
                   ln_g.reshape(1, D), ln_b.reshape(1, D), mask)
    return out
